# Initial kernel scaffold; baseline (speedup 1.0000x reference)
import jax
import jax.numpy as jnp
from jax.experimental import pallas as pl


def kernel(x, batch):
    raise NotImplementedError("write your pallas kernel here")


# speedup vs baseline: 4.8931x; 4.8931x over previous
import functools

import jax
import jax.numpy as jnp
from jax import lax
from jax.experimental import pallas as pl
from jax.experimental.pallas import tpu as pltpu

K = 20
D = 3
N = 8192
R = 128
CW = 256
NCH = N // CW
BIG = 2**30


def _knn_kernel(xr_ref, xt_ref, brow_ref, bcol_ref, out_ref, dist_ref):
    px = xr_ref[:, 0:1]
    py = xr_ref[:, 1:2]
    pz = xr_ref[:, 2:3]
    sqr = (px * px + py * py) + pz * pz
    pxb = px.astype(jnp.bfloat16).astype(jnp.float32)
    pyb = py.astype(jnp.bfloat16).astype(jnp.float32)
    pzb = pz.astype(jnp.bfloat16).astype(jnp.float32)
    brow = brow_ref[...]

    bmin = brow_ref[0, 0]
    bmax = brow_ref[R - 1, 0]
    bcol = bcol_ref[...]

    s = jnp.sum((bcol < bmin).astype(jnp.int32))
    e = jnp.sum((bcol <= bmax).astype(jnp.int32))
    c0 = s // CW
    c1 = (e + CW - 1) // CW

    def dist_body(c, _):
        off = pl.multiple_of(c * CW, CW)
        qx = xt_ref[0:1, pl.ds(off, CW)]
        qy = xt_ref[1:2, pl.ds(off, CW)]
        qz = xt_ref[2:3, pl.ds(off, CW)]
        bc = bcol_ref[0:1, pl.ds(off, CW)]
        sqc = (qx * qx + qy * qy) + qz * qz
        qxb = qx.astype(jnp.bfloat16).astype(jnp.float32)
        qyb = qy.astype(jnp.bfloat16).astype(jnp.float32)
        qzb = qz.astype(jnp.bfloat16).astype(jnp.float32)
        dot = (pxb * qxb + pyb * qyb) + pzb * qzb
        d = (sqr + sqc) - 2.0 * dot
        d = jnp.where(brow == bc, d, jnp.inf)
        dist_ref[:, pl.ds(off, CW)] = d
        return 0

    lax.fori_loop(c0, c1, dist_body, 0)

    for t in range(K):
        def pass1(c, carry):
            mval, midx = carry
            off = pl.multiple_of(c * CW, CW)
            d = dist_ref[:, pl.ds(off, CW)]
            cmin = jnp.min(d, axis=1, keepdims=True)
            ii = lax.broadcasted_iota(jnp.int32, (R, CW), 1) + off
            cidx = jnp.min(jnp.where(d == cmin, ii, jnp.int32(BIG)), axis=1,
                           keepdims=True)
            upd = cmin < mval
            return (jnp.where(upd, cmin, mval), jnp.where(upd, cidx, midx))

        mval0 = jnp.full((R, 1), jnp.inf, jnp.float32)
        midx0 = jnp.full((R, 1), BIG, jnp.int32)
        _, midx = lax.fori_loop(c0, c1, pass1, (mval0, midx0))
        out_ref[:, t:t + 1] = midx

        def pass2(c, _):
            off = pl.multiple_of(c * CW, CW)
            d = dist_ref[:, pl.ds(off, CW)]
            ii = lax.broadcasted_iota(jnp.int32, (R, CW), 1) + off
            dist_ref[:, pl.ds(off, CW)] = jnp.where(ii == midx, jnp.inf, d)
            return 0

        lax.fori_loop(c0, c1, pass2, 0)


def kernel(x, batch):
    xr = x.astype(jnp.float32)
    xt = xr.T
    b32 = batch.astype(jnp.int32)
    brow = b32.reshape(N, 1)
    bcol = b32.reshape(1, N)

    grid = (N // R,)
    out = pl.pallas_call(
        _knn_kernel,
        grid=grid,
        in_specs=[
            pl.BlockSpec((R, 8), lambda g: (g, 0)),
            pl.BlockSpec((8, N), lambda g: (0, 0)),
            pl.BlockSpec((R, 1), lambda g: (g, 0)),
            pl.BlockSpec((1, N), lambda g: (0, 0)),
        ],
        out_specs=pl.BlockSpec((R, K), lambda g: (g, 0)),
        out_shape=jax.ShapeDtypeStruct((N, K), jnp.int32),
        scratch_shapes=[pltpu.VMEM((R, N), jnp.float32)],
    )(xr, xt, brow, bcol)
    return out


# speedup vs baseline: 12.3343x; 2.5208x over previous
import jax
import jax.numpy as jnp
from jax import lax
from jax.experimental import pallas as pl
from jax.experimental.pallas import tpu as pltpu

K = 20
N = 8192
R = 256
CW = 512
BIG = 2**30


def _knn_kernel(xr_ref, xt_ref, brow_ref, bcol_ref, out_ref, dist_ref):
    px = xr_ref[:, 0:1]
    py = xr_ref[:, 1:2]
    pz = xr_ref[:, 2:3]
    sqr = (px * px + py * py) + pz * pz
    pxb = px.astype(jnp.bfloat16).astype(jnp.float32)
    pyb = py.astype(jnp.bfloat16).astype(jnp.float32)
    pzb = pz.astype(jnp.bfloat16).astype(jnp.float32)
    brow = brow_ref[...]

    bmin = brow_ref[0, 0]
    bmax = brow_ref[R - 1, 0]
    bcol = bcol_ref[...]

    s = jnp.sum((bcol < bmin).astype(jnp.int32))
    e = jnp.sum((bcol <= bmax).astype(jnp.int32))
    c0 = s // CW
    c1 = (e + CW - 1) // CW

    prev = jnp.full((R, 1), -1, jnp.int32)
    for t in range(K):
        def scan(c, carry, prev=prev, t=t):
            mval, midx = carry
            off = pl.multiple_of(c * CW, CW)
            ii = lax.broadcasted_iota(jnp.int32, (R, CW), 1) + off
            if t == 0:
                qx = xt_ref[0:1, pl.ds(off, CW)]
                qy = xt_ref[1:2, pl.ds(off, CW)]
                qz = xt_ref[2:3, pl.ds(off, CW)]
                bc = bcol_ref[0:1, pl.ds(off, CW)]
                sqc = (qx * qx + qy * qy) + qz * qz
                qxb = qx.astype(jnp.bfloat16).astype(jnp.float32)
                qyb = qy.astype(jnp.bfloat16).astype(jnp.float32)
                qzb = qz.astype(jnp.bfloat16).astype(jnp.float32)
                dot = (pxb * qxb + pyb * qyb) + pzb * qzb
                d = (sqr + sqc) - 2.0 * dot
                d = jnp.where(brow == bc, d, jnp.inf)
                dist_ref[:, pl.ds(off, CW)] = d
            else:
                d = dist_ref[:, pl.ds(off, CW)]
                d = jnp.where(ii == prev, jnp.inf, d)
                if t < K - 1:
                    dist_ref[:, pl.ds(off, CW)] = d
            cmin = jnp.min(d, axis=1, keepdims=True)
            cidx = jnp.min(jnp.where(d == cmin, ii, jnp.int32(BIG)),
                           axis=1, keepdims=True)
            upd = cmin < mval
            return (jnp.where(upd, cmin, mval), jnp.where(upd, cidx, midx))

        mval0 = jnp.full((R, 1), jnp.inf, jnp.float32)
        midx0 = jnp.full((R, 1), BIG, jnp.int32)
        _, midx = lax.fori_loop(c0, c1, scan, (mval0, midx0))
        out_ref[:, t:t + 1] = midx
        prev = midx


def kernel(x, batch):
    xr = x.astype(jnp.float32)
    xt = xr.T
    b32 = batch.astype(jnp.int32)
    brow = b32.reshape(N, 1)
    bcol = b32.reshape(1, N)

    grid = (N // R,)
    out = pl.pallas_call(
        _knn_kernel,
        grid=grid,
        in_specs=[
            pl.BlockSpec((R, 8), lambda g: (g, 0)),
            pl.BlockSpec((8, N), lambda g: (0, 0)),
            pl.BlockSpec((R, 1), lambda g: (g, 0)),
            pl.BlockSpec((1, N), lambda g: (0, 0)),
        ],
        out_specs=pl.BlockSpec((R, K), lambda g: (g, 0)),
        out_shape=jax.ShapeDtypeStruct((N, K), jnp.int32),
        scratch_shapes=[pltpu.VMEM((R, N), jnp.float32)],
    )(xr, xt, brow, bcol)
    return out


# speedup vs baseline: 13.3664x; 1.0837x over previous
import jax
import jax.numpy as jnp
from jax import lax
from jax.experimental import pallas as pl
from jax.experimental.pallas import tpu as pltpu
from jax.experimental.pallas import tpu_sc as plsc

K = 20
N = 8192
NW = 32
RPW = N // NW
OUTW = 32
L = 16
NCH = N // L
BIGI = 2**30
INF = float("inf")


def _iota():
    return lax.iota(jnp.int32, L)


def _rne_bf16(v):
    bits = lax.bitcast_convert_type(v, jnp.uint32)
    r = bits + jnp.uint32(0x7FFF) + ((bits >> jnp.uint32(16)) & jnp.uint32(1))
    r = r & jnp.uint32(0xFFFF0000)
    return lax.bitcast_convert_type(r, jnp.float32)


def _dyn_gather(vec, idx):
    dnums = lax.GatherDimensionNumbers(
        offset_dims=(), collapsed_slice_dims=(0,), start_index_map=(0,))
    return lax.gather(vec, idx[:, None], dnums, (1,),
                      mode=lax.GatherScatterMode.PROMISE_IN_BOUNDS)


def _scalar(ref, idx):
    return plsc.load_gather(ref, [jnp.full((L,), idx, jnp.int32)])[0]


def _body(pxh, pyh, pzh, bh, outh,
          pxv, pyv, pzv, bv, xbv, ybv, zbv, sqv, srv, erv, dbuf, cmbuf, outv):
    wid = lax.axis_index("s") * 2 + lax.axis_index("c")

    pltpu.sync_copy(pxh, pxv)
    pltpu.sync_copy(pyh, pyv)
    pltpu.sync_copy(pzh, pzv)
    pltpu.sync_copy(bh, bv.at[pl.ds(0, N)])

    lbs = []
    for b in range(1, 9):
        lo = jnp.int32(0)
        for s in (4096, 2048, 1024, 512, 256, 128, 64, 32, 16, 8, 4, 2, 1):
            cand = lo + s
            v = _scalar(bv, cand - 1)
            lo = jnp.where(v < b, cand, lo)
        lbs.append(lo)
    start_vec = jnp.zeros((L,), jnp.int32)
    end_vec = jnp.full((L,), N, jnp.int32)
    for b in range(1, 9):
        start_vec = jnp.where(_iota() == b, lbs[b - 1], start_vec)
    for b in range(0, 8):
        end_vec = jnp.where(_iota() == b, lbs[b], end_vec)

    def stage(j, _):
        off = pl.multiple_of(j * L, L)
        qx = pxv[pl.ds(off, L)]
        qy = pyv[pl.ds(off, L)]
        qz = pzv[pl.ds(off, L)]
        sqv[pl.ds(off, L)] = (qx * qx + qy * qy) + qz * qz
        xbv[pl.ds(off, L)] = _rne_bf16(qx)
        ybv[pl.ds(off, L)] = _rne_bf16(qy)
        zbv[pl.ds(off, L)] = _rne_bf16(qz)
        bc = bv[pl.ds(off, L)]
        srv[pl.ds(off, L)] = _dyn_gather(start_vec, bc)
        erv[pl.ds(off, L)] = _dyn_gather(end_vec, bc)
        return 0

    lax.fori_loop(0, NCH, stage, 0)

    r0 = wid * RPW

    def row_body(r, _):
        sr = _scalar(srv, r)
        er = _scalar(erv, r)
        rsq = _scalar(sqv, r)
        rxb = _scalar(xbv, r)
        ryb = _scalar(ybv, r)
        rzb = _scalar(zbv, r)
        sc0 = sr >> 8
        sc1 = (er + 255) >> 8

        def phase_a(s, carry):
            mcm, pcm = carry
            cmvec = jnp.full((L,), INF, jnp.float32)
            base = pl.multiple_of(s * 256, 256)
            for u in range(16):
                off = base + u * L
                qx = xbv[pl.ds(off, L)]
                qy = ybv[pl.ds(off, L)]
                qz = zbv[pl.ds(off, L)]
                sqc = sqv[pl.ds(off, L)]
                dot = (rxb * qx + ryb * qy) + rzb * qz
                d = (rsq + sqc) - 2.0 * dot
                ii = off + _iota()
                d = jnp.where((ii >= sr) & (ii < er), d, INF)
                dbuf[pl.ds(off, L)] = d
                cmvec = jnp.where(_iota() == u, jnp.min(d), cmvec)
            cmbuf[pl.ds(pl.multiple_of(s * L, L), L)] = cmvec
            cpos = s * L + _iota()
            upd = cmvec < mcm
            mcm = jnp.where(upd, cmvec, mcm)
            pcm = jnp.where(upd, cpos, pcm)
            return mcm, pcm

        mcm0 = jnp.full((L,), INF, jnp.float32)
        pcm0 = jnp.full((L,), BIGI, jnp.int32)
        mcm, pcm = lax.fori_loop(sc0, sc1, phase_a, (mcm0, pcm0))

        def extract(t, carry):
            mcm, pcm, a0, a1 = carry
            g = jnp.min(mcm)
            cstar = jnp.min(jnp.where(mcm == g, pcm, BIGI))
            cidx = cstar * L + _iota()
            dd = plsc.load_gather(dbuf, [cidx])
            gp = jnp.min(jnp.where(dd == g, cidx, BIGI))
            a0 = jnp.where(_iota() == t, gp, a0)
            a1 = jnp.where(_iota() == (t - 16), gp, a1)
            dd = jnp.where(_iota() == (gp & 15), INF, dd)
            plsc.store_scatter(dbuf, [cidx], dd)
            nc = jnp.min(dd)
            plsc.store_scatter(cmbuf, [jnp.full((L,), cstar, jnp.int32)],
                               jnp.full((L,), nc, jnp.float32),
                               mask=_iota() == 0)

            def rescan(s, rc):
                rm, rp = rc
                cvs = cmbuf[pl.ds(pl.multiple_of(s * L, L), L)]
                cps = s * L + _iota()
                upd = cvs < rm
                return (jnp.where(upd, cvs, rm), jnp.where(upd, cps, rp))

            rm0 = jnp.full((L,), INF, jnp.float32)
            rp0 = jnp.full((L,), BIGI, jnp.int32)
            mcm, pcm = lax.fori_loop(sc0, sc1, rescan, (rm0, rp0))
            return mcm, pcm, a0, a1

        a0 = jnp.zeros((L,), jnp.int32)
        a1 = jnp.zeros((L,), jnp.int32)
        _, _, a0, a1 = lax.fori_loop(0, K, extract, (mcm, pcm, a0, a1))

        roff = pl.multiple_of((r - r0) * OUTW, OUTW)
        outv[pl.ds(roff, L)] = a0
        outv[pl.ds(roff + L, L)] = a1
        return 0

    lax.fori_loop(r0, r0 + RPW, row_body, 0)

    pltpu.sync_copy(outv, outh.at[pl.ds(wid * RPW * OUTW, RPW * OUTW)])


def _sc_call(px, py, pz, b32):
    mesh = plsc.VectorSubcoreMesh(core_axis_name="c", subcore_axis_name="s",
                                  num_cores=2, num_subcores=16)
    fn = pl.kernel(
        _body,
        out_type=jax.ShapeDtypeStruct((N * OUTW,), jnp.int32),
        mesh=mesh,
        compiler_params=pltpu.CompilerParams(needs_layout_passes=False),
        scratch_types=[
            pltpu.VMEM((N,), jnp.float32),
            pltpu.VMEM((N,), jnp.float32),
            pltpu.VMEM((N,), jnp.float32),
            pltpu.VMEM((N + L,), jnp.int32),
            pltpu.VMEM((N + L,), jnp.float32),
            pltpu.VMEM((N + L,), jnp.float32),
            pltpu.VMEM((N + L,), jnp.float32),
            pltpu.VMEM((N + L,), jnp.float32),
            pltpu.VMEM((N + L,), jnp.int32),
            pltpu.VMEM((N + L,), jnp.int32),
            pltpu.VMEM((N,), jnp.float32),
            pltpu.VMEM((NCH,), jnp.float32),
            pltpu.VMEM((RPW * OUTW,), jnp.int32),
        ],
    )
    return fn(px, py, pz, b32)


def kernel(x, batch):
    xf = x.astype(jnp.float32)
    px = xf[:, 0]
    py = xf[:, 1]
    pz = xf[:, 2]
    b32 = batch.astype(jnp.int32)
    out = _sc_call(px, py, pz, b32)
    return out.reshape(N, OUTW)[:, :K]


# speedup vs baseline: 13.6173x; 1.0188x over previous
import jax
import jax.numpy as jnp
from jax import lax
from jax.experimental import pallas as pl
from jax.experimental.pallas import tpu as pltpu
from jax.experimental.pallas import tpu_sc as plsc

K = 20
N = 8192
NW = 32
RPW = N // NW
OUTW = 32
L = 16
NCH = N // L
BIGI = 2**30
INF = float("inf")


def _iota():
    return lax.iota(jnp.int32, L)


def _rne_bf16(v):
    bits = lax.bitcast_convert_type(v, jnp.uint32)
    r = bits + jnp.uint32(0x7FFF) + ((bits >> jnp.uint32(16)) & jnp.uint32(1))
    r = r & jnp.uint32(0xFFFF0000)
    return lax.bitcast_convert_type(r, jnp.float32)


def _dyn_gather(vec, idx):
    dnums = lax.GatherDimensionNumbers(
        offset_dims=(), collapsed_slice_dims=(0,), start_index_map=(0,))
    return lax.gather(vec, idx[:, None], dnums, (1,),
                      mode=lax.GatherScatterMode.PROMISE_IN_BOUNDS)


def _vmin_all(v):
    for s in (8, 4, 2, 1):
        v = jnp.minimum(v, _dyn_gather(v, _iota() ^ s))
    return v


def _scalar(ref, idx):
    return plsc.load_gather(ref, [jnp.full((L,), idx, jnp.int32)])[0]


def _body(pxh, pyh, pzh, bh, outh,
          pxv, pyv, pzv, bv, xbv, ybv, zbv, sqv, srv, erv, dbuf, cmbuf, outv):
    wid = lax.axis_index("s") * 2 + lax.axis_index("c")

    pltpu.sync_copy(pxh, pxv)
    pltpu.sync_copy(pyh, pyv)
    pltpu.sync_copy(pzh, pzv)
    pltpu.sync_copy(bh, bv.at[pl.ds(0, N)])

    lbs = []
    for b in range(1, 9):
        lo = jnp.int32(0)
        for s in (4096, 2048, 1024, 512, 256, 128, 64, 32, 16, 8, 4, 2, 1):
            cand = lo + s
            v = _scalar(bv, cand - 1)
            lo = jnp.where(v < b, cand, lo)
        lbs.append(lo)
    start_vec = jnp.zeros((L,), jnp.int32)
    end_vec = jnp.full((L,), N, jnp.int32)
    for b in range(1, 9):
        start_vec = jnp.where(_iota() == b, lbs[b - 1], start_vec)
    for b in range(0, 8):
        end_vec = jnp.where(_iota() == b, lbs[b], end_vec)

    def stage(j, _):
        off = pl.multiple_of(j * L, L)
        qx = pxv[pl.ds(off, L)]
        qy = pyv[pl.ds(off, L)]
        qz = pzv[pl.ds(off, L)]
        sqv[pl.ds(off, L)] = (qx * qx + qy * qy) + qz * qz
        xbv[pl.ds(off, L)] = _rne_bf16(qx)
        ybv[pl.ds(off, L)] = _rne_bf16(qy)
        zbv[pl.ds(off, L)] = _rne_bf16(qz)
        bc = bv[pl.ds(off, L)]
        srv[pl.ds(off, L)] = _dyn_gather(start_vec, bc)
        erv[pl.ds(off, L)] = _dyn_gather(end_vec, bc)
        return 0

    lax.fori_loop(0, NCH, stage, 0)

    r0 = wid * RPW

    def row_body(r, _):
        sr = _scalar(srv, r)
        er = _scalar(erv, r)
        rsq = _scalar(sqv, r)
        rxb = _scalar(xbv, r)
        ryb = _scalar(ybv, r)
        rzb = _scalar(zbv, r)
        sc0 = sr >> 8
        sc1 = (er + 255) >> 8

        def phase_a(s, carry):
            mcm, pcm = carry
            cmvec = jnp.full((L,), INF, jnp.float32)
            base = pl.multiple_of(s * 256, 256)
            for u in range(16):
                off = base + u * L
                qx = xbv[pl.ds(off, L)]
                qy = ybv[pl.ds(off, L)]
                qz = zbv[pl.ds(off, L)]
                sqc = sqv[pl.ds(off, L)]
                dot = (rxb * qx + ryb * qy) + rzb * qz
                d = (rsq + sqc) - 2.0 * dot
                ii = off + _iota()
                d = jnp.where((ii >= sr) & (ii < er), d, INF)
                dbuf[pl.ds(off, L)] = d
                cmvec = jnp.where(_iota() == u, _vmin_all(d), cmvec)
            cmbuf[pl.ds(pl.multiple_of(s * L, L), L)] = cmvec
            cpos = s * L + _iota()
            upd = cmvec < mcm
            mcm = jnp.where(upd, cmvec, mcm)
            pcm = jnp.where(upd, cpos, pcm)
            return mcm, pcm

        mcm0 = jnp.full((L,), INF, jnp.float32)
        pcm0 = jnp.full((L,), BIGI, jnp.int32)
        mcm, pcm = lax.fori_loop(sc0, sc1, phase_a, (mcm0, pcm0))

        def extract(t, carry):
            mcm, pcm, a0, a1 = carry
            g = _vmin_all(mcm)
            cstar = _vmin_all(jnp.where(mcm == g, pcm, BIGI))
            cidx = cstar * L + _iota()
            dd = plsc.load_gather(dbuf, [cidx])
            gp = _vmin_all(jnp.where(dd == g, cidx, BIGI))
            a0 = jnp.where(_iota() == t, gp, a0)
            a1 = jnp.where(_iota() == (t - 16), gp, a1)
            dd = jnp.where(cidx == gp, INF, dd)
            plsc.store_scatter(dbuf, [cidx], dd)
            nc = _vmin_all(dd)
            plsc.store_scatter(cmbuf, [cstar], nc, mask=_iota() == 0)

            def rescan(s, rc):
                rm, rp = rc
                cvs = cmbuf[pl.ds(pl.multiple_of(s * L, L), L)]
                cps = s * L + _iota()
                upd = cvs < rm
                return (jnp.where(upd, cvs, rm), jnp.where(upd, cps, rp))

            rm0 = jnp.full((L,), INF, jnp.float32)
            rp0 = jnp.full((L,), BIGI, jnp.int32)
            mcm, pcm = lax.fori_loop(sc0, sc1, rescan, (rm0, rp0))
            return mcm, pcm, a0, a1

        a0 = jnp.zeros((L,), jnp.int32)
        a1 = jnp.zeros((L,), jnp.int32)
        _, _, a0, a1 = lax.fori_loop(0, K, extract, (mcm, pcm, a0, a1))

        roff = pl.multiple_of((r - r0) * OUTW, OUTW)
        outv[pl.ds(roff, L)] = a0
        outv[pl.ds(roff + L, L)] = a1
        return 0

    lax.fori_loop(r0, r0 + RPW, row_body, 0)

    pltpu.sync_copy(outv, outh.at[pl.ds(wid * RPW * OUTW, RPW * OUTW)])


def _sc_call(px, py, pz, b32):
    mesh = plsc.VectorSubcoreMesh(core_axis_name="c", subcore_axis_name="s",
                                  num_cores=2, num_subcores=16)
    fn = pl.kernel(
        _body,
        out_type=jax.ShapeDtypeStruct((N * OUTW,), jnp.int32),
        mesh=mesh,
        compiler_params=pltpu.CompilerParams(needs_layout_passes=False),
        scratch_types=[
            pltpu.VMEM((N,), jnp.float32),
            pltpu.VMEM((N,), jnp.float32),
            pltpu.VMEM((N,), jnp.float32),
            pltpu.VMEM((N + L,), jnp.int32),
            pltpu.VMEM((N + L,), jnp.float32),
            pltpu.VMEM((N + L,), jnp.float32),
            pltpu.VMEM((N + L,), jnp.float32),
            pltpu.VMEM((N + L,), jnp.float32),
            pltpu.VMEM((N + L,), jnp.int32),
            pltpu.VMEM((N + L,), jnp.int32),
            pltpu.VMEM((N,), jnp.float32),
            pltpu.VMEM((NCH,), jnp.float32),
            pltpu.VMEM((RPW * OUTW,), jnp.int32),
        ],
    )
    return fn(px, py, pz, b32)


def kernel(x, batch):
    xf = x.astype(jnp.float32)
    px = xf[:, 0]
    py = xf[:, 1]
    pz = xf[:, 2]
    b32 = batch.astype(jnp.int32)
    out = _sc_call(px, py, pz, b32)
    return out.reshape(N, OUTW)[:, :K]


# speedup vs baseline: 14.9369x; 1.0969x over previous
import jax
import jax.numpy as jnp
from jax import lax
from jax.experimental import pallas as pl
from jax.experimental.pallas import tpu as pltpu
from jax.experimental.pallas import tpu_sc as plsc

K = 20
N = 8192
NW = 32
RPW = N // NW
OUTW = 32
L = 16
NCH = N // L
BIGI = 2**30
INF = float("inf")


def _iota():
    return lax.iota(jnp.int32, L)


def _rne_bf16(v):
    bits = lax.bitcast_convert_type(v, jnp.uint32)
    r = bits + jnp.uint32(0x7FFF) + ((bits >> jnp.uint32(16)) & jnp.uint32(1))
    r = r & jnp.uint32(0xFFFF0000)
    return lax.bitcast_convert_type(r, jnp.float32)


def _dyn_gather(vec, idx):
    dnums = lax.GatherDimensionNumbers(
        offset_dims=(), collapsed_slice_dims=(0,), start_index_map=(0,))
    return lax.gather(vec, idx[:, None], dnums, (1,),
                      mode=lax.GatherScatterMode.PROMISE_IN_BOUNDS)


def _vmin_all(v):
    for s in (8, 4, 2, 1):
        v = jnp.minimum(v, _dyn_gather(v, _iota() ^ s))
    return v


def _scalar(ref, idx):
    return plsc.load_gather(ref, [jnp.full((L,), idx, jnp.int32)])[0]


def _body(pxh, pyh, pzh, bh, outh,
          pxv, pyv, pzv, bv, xbv, ybv, zbv, sqv, srv, erv, dbuf, cmbuf, outv):
    wid = lax.axis_index("s") * 2 + lax.axis_index("c")

    pltpu.sync_copy(pxh, pxv)
    pltpu.sync_copy(pyh, pyv)
    pltpu.sync_copy(pzh, pzv)
    pltpu.sync_copy(bh, bv.at[pl.ds(0, N)])

    lbs = []
    for b in range(1, 9):
        lo = jnp.int32(0)
        for s in (4096, 2048, 1024, 512, 256, 128, 64, 32, 16, 8, 4, 2, 1):
            cand = lo + s
            v = _scalar(bv, cand - 1)
            lo = jnp.where(v < b, cand, lo)
        lbs.append(lo)
    start_vec = jnp.zeros((L,), jnp.int32)
    end_vec = jnp.full((L,), N, jnp.int32)
    for b in range(1, 9):
        start_vec = jnp.where(_iota() == b, lbs[b - 1], start_vec)
    for b in range(0, 8):
        end_vec = jnp.where(_iota() == b, lbs[b], end_vec)

    def stage(j, _):
        off = pl.multiple_of(j * L, L)
        qx = pxv[pl.ds(off, L)]
        qy = pyv[pl.ds(off, L)]
        qz = pzv[pl.ds(off, L)]
        sqv[pl.ds(off, L)] = (qx * qx + qy * qy) + qz * qz
        xbv[pl.ds(off, L)] = _rne_bf16(qx)
        ybv[pl.ds(off, L)] = _rne_bf16(qy)
        zbv[pl.ds(off, L)] = _rne_bf16(qz)
        bc = bv[pl.ds(off, L)]
        srv[pl.ds(off, L)] = _dyn_gather(start_vec, bc)
        erv[pl.ds(off, L)] = _dyn_gather(end_vec, bc)
        return 0

    lax.fori_loop(0, NCH, stage, 0)

    r0 = wid * RPW

    def row_body(r, _):
        sr = _scalar(srv, r)
        er = _scalar(erv, r)
        rsq = _scalar(sqv, r)
        rxb = _scalar(xbv, r)
        ryb = _scalar(ybv, r)
        rzb = _scalar(zbv, r)
        sc0 = sr >> 8
        sc1 = (er + 255) >> 8

        def phase_a(s, carry):
            mcm, pcm = carry
            cmvec = jnp.full((L,), INF, jnp.float32)
            base = pl.multiple_of(s * 256, 256)
            for u in range(16):
                off = base + u * L
                qx = xbv[pl.ds(off, L)]
                qy = ybv[pl.ds(off, L)]
                qz = zbv[pl.ds(off, L)]
                sqc = sqv[pl.ds(off, L)]
                dot = (rxb * qx + ryb * qy) + rzb * qz
                d = (rsq + sqc) - 2.0 * dot
                ii = off + _iota()
                d = jnp.where((ii >= sr) & (ii < er), d, INF)
                dbuf[pl.ds(off, L)] = d
                cmvec = jnp.where(_iota() == u, _vmin_all(d), cmvec)
            cmbuf[pl.ds(pl.multiple_of(s * L, L), L)] = cmvec
            cpos = s * L + _iota()
            upd = cmvec < mcm
            mcm = jnp.where(upd, cmvec, mcm)
            pcm = jnp.where(upd, cpos, pcm)
            return mcm, pcm

        mcm0 = jnp.full((L,), INF, jnp.float32)
        pcm0 = jnp.full((L,), BIGI, jnp.int32)
        mcm, pcm = lax.fori_loop(sc0, sc1, phase_a, (mcm0, pcm0))

        def extract(t, carry):
            mcm, pcm, a0, a1 = carry
            g = _vmin_all(mcm)
            cstar = _vmin_all(jnp.where(mcm == g, pcm, BIGI))
            cidx = cstar * L + _iota()
            dd = plsc.load_gather(dbuf, [cidx])
            gp = _vmin_all(jnp.where(dd == g, cidx, BIGI))
            a0 = jnp.where(_iota() == t, gp, a0)
            a1 = jnp.where(_iota() == (t - 16), gp, a1)
            dd = jnp.where(cidx == gp, INF, dd)
            plsc.store_scatter(dbuf, [cidx], dd)
            nc = _vmin_all(dd)
            plsc.store_scatter(cmbuf, [cstar], nc, mask=_iota() == 0)

            lcm = cstar & 15
            rm = jnp.full((L,), INF, jnp.float32)
            rpos = jnp.full((L,), BIGI, jnp.int32)
            for gi in range(2):
                scs = sc0 + gi * L + _iota()
                ok = scs < sc1
                idx = jnp.where(ok, scs * L + lcm, 0)
                col = plsc.load_gather(cmbuf, [idx])
                col = jnp.where(ok, col, INF)
                upd = col < rm
                rm = jnp.where(upd, col, rm)
                rpos = jnp.where(upd, idx, rpos)
            newm = _vmin_all(rm)
            newp = _vmin_all(jnp.where(rm == newm, rpos, BIGI))
            lmask = _iota() == lcm
            mcm = jnp.where(lmask, newm, mcm)
            pcm = jnp.where(lmask, newp, pcm)
            return mcm, pcm, a0, a1

        a0 = jnp.zeros((L,), jnp.int32)
        a1 = jnp.zeros((L,), jnp.int32)
        _, _, a0, a1 = lax.fori_loop(0, K, extract, (mcm, pcm, a0, a1))

        roff = pl.multiple_of((r - r0) * OUTW, OUTW)
        outv[pl.ds(roff, L)] = a0
        outv[pl.ds(roff + L, L)] = a1
        return 0

    lax.fori_loop(r0, r0 + RPW, row_body, 0)

    pltpu.sync_copy(outv, outh.at[pl.ds(wid * RPW * OUTW, RPW * OUTW)])


def _sc_call(px, py, pz, b32):
    mesh = plsc.VectorSubcoreMesh(core_axis_name="c", subcore_axis_name="s",
                                  num_cores=2, num_subcores=16)
    fn = pl.kernel(
        _body,
        out_type=jax.ShapeDtypeStruct((N * OUTW,), jnp.int32),
        mesh=mesh,
        compiler_params=pltpu.CompilerParams(needs_layout_passes=False),
        scratch_types=[
            pltpu.VMEM((N,), jnp.float32),
            pltpu.VMEM((N,), jnp.float32),
            pltpu.VMEM((N,), jnp.float32),
            pltpu.VMEM((N + L,), jnp.int32),
            pltpu.VMEM((N + L,), jnp.float32),
            pltpu.VMEM((N + L,), jnp.float32),
            pltpu.VMEM((N + L,), jnp.float32),
            pltpu.VMEM((N + L,), jnp.float32),
            pltpu.VMEM((N + L,), jnp.int32),
            pltpu.VMEM((N + L,), jnp.int32),
            pltpu.VMEM((N,), jnp.float32),
            pltpu.VMEM((NCH,), jnp.float32),
            pltpu.VMEM((RPW * OUTW,), jnp.int32),
        ],
    )
    return fn(px, py, pz, b32)


def kernel(x, batch):
    xf = x.astype(jnp.float32)
    px = xf[:, 0]
    py = xf[:, 1]
    pz = xf[:, 2]
    b32 = batch.astype(jnp.int32)
    out = _sc_call(px, py, pz, b32)
    return out.reshape(N, OUTW)[:, :K]


# speedup vs baseline: 27.3331x; 1.8299x over previous
import jax
import jax.numpy as jnp
from jax import lax
from jax.experimental import pallas as pl
from jax.experimental.pallas import tpu as pltpu
from jax.experimental.pallas import tpu_sc as plsc

K = 20
N = 8192
NW = 32
RPW = N // NW
OUTW = 32
L = 16
NCH = N // L
BIGI = 2**30
INF = float("inf")


def _iota():
    return lax.iota(jnp.int32, L)


def _rne_bf16(v):
    bits = lax.bitcast_convert_type(v, jnp.uint32)
    r = bits + jnp.uint32(0x7FFF) + ((bits >> jnp.uint32(16)) & jnp.uint32(1))
    r = r & jnp.uint32(0xFFFF0000)
    return lax.bitcast_convert_type(r, jnp.float32)


def _dyn_gather(vec, idx):
    dnums = lax.GatherDimensionNumbers(
        offset_dims=(), collapsed_slice_dims=(0,), start_index_map=(0,))
    return lax.gather(vec, idx[:, None], dnums, (1,),
                      mode=lax.GatherScatterMode.PROMISE_IN_BOUNDS)


def _vmin_all(v):
    for s in (8, 4, 2, 1):
        v = jnp.minimum(v, _dyn_gather(v, _iota() ^ s))
    return v


def _scalar(ref, idx):
    return plsc.load_gather(ref, [jnp.full((L,), idx, jnp.int32)])[0]


def _body(pxh, pyh, pzh, bh, outh,
          pxv, pyv, pzv, bv, xbv, ybv, zbv, sqv, srv, erv, dbuf, cmbuf, outv):
    wid = lax.axis_index("s") * 2 + lax.axis_index("c")

    pltpu.sync_copy(pxh, pxv)
    pltpu.sync_copy(pyh, pyv)
    pltpu.sync_copy(pzh, pzv)
    pltpu.sync_copy(bh, bv.at[pl.ds(0, N)])

    lbs = []
    for b in range(1, 9):
        lo = jnp.int32(0)
        for s in (4096, 2048, 1024, 512, 256, 128, 64, 32, 16, 8, 4, 2, 1):
            cand = lo + s
            v = _scalar(bv, cand - 1)
            lo = jnp.where(v < b, cand, lo)
        lbs.append(lo)
    start_vec = jnp.zeros((L,), jnp.int32)
    end_vec = jnp.full((L,), N, jnp.int32)
    for b in range(1, 9):
        start_vec = jnp.where(_iota() == b, lbs[b - 1], start_vec)
    for b in range(0, 8):
        end_vec = jnp.where(_iota() == b, lbs[b], end_vec)

    def stage(j, _):
        off = pl.multiple_of(j * L, L)
        qx = pxv[pl.ds(off, L)]
        qy = pyv[pl.ds(off, L)]
        qz = pzv[pl.ds(off, L)]
        sqv[pl.ds(off, L)] = (qx * qx + qy * qy) + qz * qz
        xbv[pl.ds(off, L)] = _rne_bf16(qx)
        ybv[pl.ds(off, L)] = _rne_bf16(qy)
        zbv[pl.ds(off, L)] = _rne_bf16(qz)
        bc = bv[pl.ds(off, L)]
        srv[pl.ds(off, L)] = _dyn_gather(start_vec, bc)
        erv[pl.ds(off, L)] = _dyn_gather(end_vec, bc)
        return 0

    lax.fori_loop(0, NCH, stage, 0)

    r0 = wid * RPW

    def row_body(i, _):
        rows = []
        for q in range(2):
            r = r0 + i * 2 + q
            rows.append((_scalar(srv, r), _scalar(erv, r), _scalar(sqv, r),
                         _scalar(xbv, r), _scalar(ybv, r), _scalar(zbv, r)))
        sc0 = rows[0][0] >> 8
        sc1 = (rows[1][1] + 255) >> 8

        def phase_a(s, carry):
            mcm = [carry[0], carry[2]]
            pcm = [carry[1], carry[3]]
            cmvec = [jnp.full((L,), INF, jnp.float32) for _ in range(2)]
            base = pl.multiple_of(s * 256, 256)
            for u in range(16):
                off = base + u * L
                qx = xbv[pl.ds(off, L)]
                qy = ybv[pl.ds(off, L)]
                qz = zbv[pl.ds(off, L)]
                sqc = sqv[pl.ds(off, L)]
                ii = off + _iota()
                for q in range(2):
                    sr, er, rsq, rxb, ryb, rzb = rows[q]
                    dot = (rxb * qx + ryb * qy) + rzb * qz
                    d = (rsq + sqc) - 2.0 * dot
                    d = jnp.where((ii >= sr) & (ii < er), d, INF)
                    dbuf[pl.ds(off + q * N, L)] = d
                    cmvec[q] = jnp.where(_iota() == u, _vmin_all(d), cmvec[q])
            cpos = s * L + _iota()
            out = []
            for q in range(2):
                cmbuf[pl.ds(pl.multiple_of(s * L, L) + q * NCH, L)] = cmvec[q]
                upd = cmvec[q] < mcm[q]
                out.append(jnp.where(upd, cmvec[q], mcm[q]))
                out.append(jnp.where(upd, cpos, pcm[q]))
            return tuple(out)

        init = (jnp.full((L,), INF, jnp.float32),
                jnp.full((L,), BIGI, jnp.int32)) * 2
        st = lax.fori_loop(sc0, sc1, phase_a, init)

        def extract(t, carry):
            out = []
            for q in range(2):
                mcm, pcm, a0, a1 = carry[q * 4:q * 4 + 4]
                g = _vmin_all(mcm)
                cstar = _vmin_all(jnp.where(mcm == g, pcm, BIGI))
                cidx = cstar * L + _iota()
                dd = plsc.load_gather(dbuf, [cidx + q * N])
                gp = _vmin_all(jnp.where(dd == g, cidx, BIGI))
                a0 = jnp.where(_iota() == t, gp, a0)
                a1 = jnp.where(_iota() == (t - 16), gp, a1)
                dd = jnp.where(cidx == gp, INF, dd)
                plsc.store_scatter(dbuf, [cidx + q * N], dd)
                nc = _vmin_all(dd)
                plsc.store_scatter(cmbuf, [cstar + q * NCH], nc,
                                   mask=_iota() == 0)

                lcm = cstar & 15
                rm = jnp.full((L,), INF, jnp.float32)
                rpos = jnp.full((L,), BIGI, jnp.int32)
                for gi in range(2):
                    scs = sc0 + gi * L + _iota()
                    ok = scs < sc1
                    idx = jnp.where(ok, scs * L + lcm, 0)
                    col = plsc.load_gather(cmbuf, [idx + q * NCH])
                    col = jnp.where(ok, col, INF)
                    upd = col < rm
                    rm = jnp.where(upd, col, rm)
                    rpos = jnp.where(upd, idx, rpos)
                newm = _vmin_all(rm)
                newp = _vmin_all(jnp.where(rm == newm, rpos, BIGI))
                lmask = _iota() == lcm
                out.append(jnp.where(lmask, newm, mcm))
                out.append(jnp.where(lmask, newp, pcm))
                out.append(a0)
                out.append(a1)
            return tuple(out)

        z = jnp.zeros((L,), jnp.int32)
        fin = lax.fori_loop(0, K, extract,
                            (st[0], st[1], z, z, st[2], st[3], z, z))

        for q in range(2):
            roff = pl.multiple_of((i * 2 + q) * OUTW, OUTW)
            outv[pl.ds(roff, L)] = fin[q * 4 + 2]
            outv[pl.ds(roff + L, L)] = fin[q * 4 + 3]
        return 0

    lax.fori_loop(0, RPW // 2, row_body, 0)

    pltpu.sync_copy(outv, outh.at[pl.ds(wid * RPW * OUTW, RPW * OUTW)])


def _sc_call(px, py, pz, b32):
    mesh = plsc.VectorSubcoreMesh(core_axis_name="c", subcore_axis_name="s",
                                  num_cores=2, num_subcores=16)
    fn = pl.kernel(
        _body,
        out_type=jax.ShapeDtypeStruct((N * OUTW,), jnp.int32),
        mesh=mesh,
        compiler_params=pltpu.CompilerParams(needs_layout_passes=False),
        scratch_types=[
            pltpu.VMEM((N,), jnp.float32),
            pltpu.VMEM((N,), jnp.float32),
            pltpu.VMEM((N,), jnp.float32),
            pltpu.VMEM((N + L,), jnp.int32),
            pltpu.VMEM((N + L,), jnp.float32),
            pltpu.VMEM((N + L,), jnp.float32),
            pltpu.VMEM((N + L,), jnp.float32),
            pltpu.VMEM((N + L,), jnp.float32),
            pltpu.VMEM((N + L,), jnp.int32),
            pltpu.VMEM((N + L,), jnp.int32),
            pltpu.VMEM((2 * N,), jnp.float32),
            pltpu.VMEM((2 * NCH,), jnp.float32),
            pltpu.VMEM((RPW * OUTW,), jnp.int32),
        ],
    )
    return fn(px, py, pz, b32)


def kernel(x, batch):
    xf = x.astype(jnp.float32)
    px = xf[:, 0]
    py = xf[:, 1]
    pz = xf[:, 2]
    b32 = batch.astype(jnp.int32)
    out = _sc_call(px, py, pz, b32)
    return out.reshape(N, OUTW)[:, :K]


# speedup vs baseline: 28.5126x; 1.0432x over previous
import jax
import jax.numpy as jnp
from jax import lax
from jax.experimental import pallas as pl
from jax.experimental.pallas import tpu as pltpu
from jax.experimental.pallas import tpu_sc as plsc

K = 20
N = 8192
NW = 32
RPW = N // NW
OUTW = 32
L = 16
NCH = N // L
G = 4
BIGI = 2**30
INF = float("inf")


def _iota():
    return lax.iota(jnp.int32, L)


def _rne_bf16(v):
    bits = lax.bitcast_convert_type(v, jnp.uint32)
    r = bits + jnp.uint32(0x7FFF) + ((bits >> jnp.uint32(16)) & jnp.uint32(1))
    r = r & jnp.uint32(0xFFFF0000)
    return lax.bitcast_convert_type(r, jnp.float32)


def _dyn_gather(vec, idx):
    dnums = lax.GatherDimensionNumbers(
        offset_dims=(), collapsed_slice_dims=(0,), start_index_map=(0,))
    return lax.gather(vec, idx[:, None], dnums, (1,),
                      mode=lax.GatherScatterMode.PROMISE_IN_BOUNDS)


def _vmin_all(v):
    for s in (8, 4, 2, 1):
        v = jnp.minimum(v, _dyn_gather(v, _iota() ^ s))
    return v


def _scalar(ref, idx):
    return plsc.load_gather(ref, [jnp.full((L,), idx, jnp.int32)])[0]


def _body(pxh, pyh, pzh, bh, outh,
          bv, xbv, ybv, zbv, sqv, srv, erv, dbuf, cmbuf, outv):
    wid = lax.axis_index("s") * 2 + lax.axis_index("c")

    pltpu.sync_copy(pxh, xbv.at[pl.ds(0, N)])
    pltpu.sync_copy(pyh, ybv.at[pl.ds(0, N)])
    pltpu.sync_copy(pzh, zbv.at[pl.ds(0, N)])
    pltpu.sync_copy(bh, bv.at[pl.ds(0, N)])

    lbs = []
    for b in range(1, 9):
        lo = jnp.int32(0)
        for s in (4096, 2048, 1024, 512, 256, 128, 64, 32, 16, 8, 4, 2, 1):
            cand = lo + s
            v = _scalar(bv, cand - 1)
            lo = jnp.where(v < b, cand, lo)
        lbs.append(lo)
    start_vec = jnp.zeros((L,), jnp.int32)
    end_vec = jnp.full((L,), N, jnp.int32)
    for b in range(1, 9):
        start_vec = jnp.where(_iota() == b, lbs[b - 1], start_vec)
    for b in range(0, 8):
        end_vec = jnp.where(_iota() == b, lbs[b], end_vec)

    def stage(j, _):
        off = pl.multiple_of(j * L, L)
        qx = xbv[pl.ds(off, L)]
        qy = ybv[pl.ds(off, L)]
        qz = zbv[pl.ds(off, L)]
        sqv[pl.ds(off, L)] = (qx * qx + qy * qy) + qz * qz
        xbv[pl.ds(off, L)] = _rne_bf16(qx)
        ybv[pl.ds(off, L)] = _rne_bf16(qy)
        zbv[pl.ds(off, L)] = _rne_bf16(qz)
        bc = bv[pl.ds(off, L)]
        srv[pl.ds(off, L)] = _dyn_gather(start_vec, bc)
        erv[pl.ds(off, L)] = _dyn_gather(end_vec, bc)
        return 0

    lax.fori_loop(0, NCH, stage, 0)

    r0 = wid * RPW

    def row_body(i, _):
        rows = []
        for q in range(G):
            r = r0 + i * G + q
            rows.append((_scalar(srv, r), _scalar(erv, r), _scalar(sqv, r),
                         _scalar(xbv, r), _scalar(ybv, r), _scalar(zbv, r)))
        sc0 = rows[0][0] >> 8
        sc1 = (rows[G - 1][1] + 255) >> 8

        def phase_a(s, carry):
            mcm = [carry[2 * q] for q in range(G)]
            pcm = [carry[2 * q + 1] for q in range(G)]
            cmvec = [jnp.full((L,), INF, jnp.float32) for _ in range(G)]
            base = pl.multiple_of(s * 256, 256)
            for u in range(16):
                off = base + u * L
                qx = xbv[pl.ds(off, L)]
                qy = ybv[pl.ds(off, L)]
                qz = zbv[pl.ds(off, L)]
                sqc = sqv[pl.ds(off, L)]
                ii = off + _iota()
                for q in range(G):
                    sr, er, rsq, rxb, ryb, rzb = rows[q]
                    dot = (rxb * qx + ryb * qy) + rzb * qz
                    d = (rsq + sqc) - 2.0 * dot
                    d = jnp.where((ii >= sr) & (ii < er), d, INF)
                    dbuf[pl.ds(off + q * N, L)] = d
                    cmvec[q] = jnp.where(_iota() == u, _vmin_all(d), cmvec[q])
            cpos = s * L + _iota()
            out = []
            for q in range(G):
                cmbuf[pl.ds(pl.multiple_of(s * L, L) + q * NCH, L)] = cmvec[q]
                upd = cmvec[q] < mcm[q]
                out.append(jnp.where(upd, cmvec[q], mcm[q]))
                out.append(jnp.where(upd, cpos, pcm[q]))
            return tuple(out)

        init = (jnp.full((L,), INF, jnp.float32),
                jnp.full((L,), BIGI, jnp.int32)) * G
        st = lax.fori_loop(sc0, sc1, phase_a, init)

        def extract(t, carry):
            out = []
            for q in range(G):
                mcm, pcm, a0, a1 = carry[q * 4:q * 4 + 4]
                g = _vmin_all(mcm)
                cstar = _vmin_all(jnp.where(mcm == g, pcm, BIGI))
                cidx = cstar * L + _iota()
                dd = plsc.load_gather(dbuf, [cidx + q * N])
                gp = _vmin_all(jnp.where(dd == g, cidx, BIGI))
                a0 = jnp.where(_iota() == t, gp, a0)
                a1 = jnp.where(_iota() == (t - 16), gp, a1)
                dd = jnp.where(cidx == gp, INF, dd)
                plsc.store_scatter(dbuf, [cidx + q * N], dd)
                nc = _vmin_all(dd)
                plsc.store_scatter(cmbuf, [cstar + q * NCH], nc,
                                   mask=_iota() == 0)

                lcm = cstar & 15
                rm = jnp.full((L,), INF, jnp.float32)
                rpos = jnp.full((L,), BIGI, jnp.int32)
                for gi in range(2):
                    scs = sc0 + gi * L + _iota()
                    ok = scs < sc1
                    idx = jnp.where(ok, scs * L + lcm, 0)
                    col = plsc.load_gather(cmbuf, [idx + q * NCH])
                    col = jnp.where(ok, col, INF)
                    upd = col < rm
                    rm = jnp.where(upd, col, rm)
                    rpos = jnp.where(upd, idx, rpos)
                newm = _vmin_all(rm)
                newp = _vmin_all(jnp.where(rm == newm, rpos, BIGI))
                lmask = _iota() == lcm
                out.append(jnp.where(lmask, newm, mcm))
                out.append(jnp.where(lmask, newp, pcm))
                out.append(a0)
                out.append(a1)
            return tuple(out)

        z = jnp.zeros((L,), jnp.int32)
        init_ex = tuple(v for q in range(G)
                        for v in (st[2 * q], st[2 * q + 1], z, z))
        fin = lax.fori_loop(0, K, extract, init_ex)

        for q in range(G):
            roff = pl.multiple_of((i * G + q) * OUTW, OUTW)
            outv[pl.ds(roff, L)] = fin[q * 4 + 2]
            outv[pl.ds(roff + L, L)] = fin[q * 4 + 3]
        return 0

    lax.fori_loop(0, RPW // G, row_body, 0)

    pltpu.sync_copy(outv, outh.at[pl.ds(wid * RPW * OUTW, RPW * OUTW)])


def _sc_call(px, py, pz, b32):
    mesh = plsc.VectorSubcoreMesh(core_axis_name="c", subcore_axis_name="s",
                                  num_cores=2, num_subcores=16)
    fn = pl.kernel(
        _body,
        out_type=jax.ShapeDtypeStruct((N * OUTW,), jnp.int32),
        mesh=mesh,
        compiler_params=pltpu.CompilerParams(needs_layout_passes=False),
        scratch_types=[
            pltpu.VMEM((N + L,), jnp.int32),
            pltpu.VMEM((N + L,), jnp.float32),
            pltpu.VMEM((N + L,), jnp.float32),
            pltpu.VMEM((N + L,), jnp.float32),
            pltpu.VMEM((N + L,), jnp.float32),
            pltpu.VMEM((N + L,), jnp.int32),
            pltpu.VMEM((N + L,), jnp.int32),
            pltpu.VMEM((G * N,), jnp.float32),
            pltpu.VMEM((G * NCH,), jnp.float32),
            pltpu.VMEM((RPW * OUTW,), jnp.int32),
        ],
    )
    return fn(px, py, pz, b32)


def kernel(x, batch):
    xf = x.astype(jnp.float32)
    px = xf[:, 0]
    py = xf[:, 1]
    pz = xf[:, 2]
    b32 = batch.astype(jnp.int32)
    out = _sc_call(px, py, pz, b32)
    return out.reshape(N, OUTW)[:, :K]


# speedup vs baseline: 32.8402x; 1.1518x over previous
import jax
import jax.numpy as jnp
from jax import lax
from jax.experimental import pallas as pl
from jax.experimental.pallas import tpu as pltpu
from jax.experimental.pallas import tpu_sc as plsc

K = 20
N = 8192
NW = 32
RPW = N // NW
OUTW = 32
L = 16
NCH = N // L
G = 4
BIGI = 2**30
INF = float("inf")


def _iota():
    return lax.iota(jnp.int32, L)


def _rne_bf16(v):
    bits = lax.bitcast_convert_type(v, jnp.uint32)
    r = bits + jnp.uint32(0x7FFF) + ((bits >> jnp.uint32(16)) & jnp.uint32(1))
    r = r & jnp.uint32(0xFFFF0000)
    return lax.bitcast_convert_type(r, jnp.float32)


def _dyn_gather(vec, idx):
    dnums = lax.GatherDimensionNumbers(
        offset_dims=(), collapsed_slice_dims=(0,), start_index_map=(0,))
    return lax.gather(vec, idx[:, None], dnums, (1,),
                      mode=lax.GatherScatterMode.PROMISE_IN_BOUNDS)


def _vmin_all(v):
    for s in (8, 4, 2, 1):
        v = jnp.minimum(v, _dyn_gather(v, _iota() ^ s))
    return v


def _scalar(ref, idx):
    return plsc.load_gather(ref, [jnp.full((L,), idx, jnp.int32)])[0]


def _body(pxh, pyh, pzh, bh, outh,
          bv, xbv, ybv, zbv, sqv, srv, erv, dbuf, cmbuf, outv):
    wid = lax.axis_index("s") * 2 + lax.axis_index("c")

    pltpu.sync_copy(pxh, xbv.at[pl.ds(0, N)])
    pltpu.sync_copy(pyh, ybv.at[pl.ds(0, N)])
    pltpu.sync_copy(pzh, zbv.at[pl.ds(0, N)])
    pltpu.sync_copy(bh, bv.at[pl.ds(0, N)])

    lbs = []
    for b in range(1, 9):
        lo = jnp.int32(0)
        for s in (4096, 2048, 1024, 512, 256, 128, 64, 32, 16, 8, 4, 2, 1):
            cand = lo + s
            v = _scalar(bv, cand - 1)
            lo = jnp.where(v < b, cand, lo)
        lbs.append(lo)
    start_vec = jnp.zeros((L,), jnp.int32)
    end_vec = jnp.full((L,), N, jnp.int32)
    for b in range(1, 9):
        start_vec = jnp.where(_iota() == b, lbs[b - 1], start_vec)
    for b in range(0, 8):
        end_vec = jnp.where(_iota() == b, lbs[b], end_vec)

    def stage(j, _):
        off = pl.multiple_of(j * L, L)
        qx = xbv[pl.ds(off, L)]
        qy = ybv[pl.ds(off, L)]
        qz = zbv[pl.ds(off, L)]
        sqv[pl.ds(off, L)] = (qx * qx + qy * qy) + qz * qz
        xbv[pl.ds(off, L)] = _rne_bf16(qx)
        ybv[pl.ds(off, L)] = _rne_bf16(qy)
        zbv[pl.ds(off, L)] = _rne_bf16(qz)
        bc = bv[pl.ds(off, L)]
        srv[pl.ds(off, L)] = _dyn_gather(start_vec, bc)
        erv[pl.ds(off, L)] = _dyn_gather(end_vec, bc)
        return 0

    lax.fori_loop(0, NCH, stage, 0)

    r0 = wid * RPW

    def row_body(i, _):
        rows = []
        for q in range(G):
            r = r0 + i * G + q
            rows.append((_scalar(srv, r), _scalar(erv, r), _scalar(sqv, r),
                         _scalar(xbv, r), _scalar(ybv, r), _scalar(zbv, r)))
        sc0 = rows[0][0] >> 8
        sc1 = (rows[G - 1][1] + 255) >> 8

        def phase_a(s, carry):
            mcm = [carry[2 * q] for q in range(G)]
            pcm = [carry[2 * q + 1] for q in range(G)]
            msc = [jnp.full((L,), INF, jnp.float32) for _ in range(G)]
            base = pl.multiple_of(s * 256, 256)
            for u in range(16):
                off = base + u * L
                qx = xbv[pl.ds(off, L)]
                qy = ybv[pl.ds(off, L)]
                qz = zbv[pl.ds(off, L)]
                sqc = sqv[pl.ds(off, L)]
                ii = off + _iota()
                for q in range(G):
                    sr, er, rsq, rxb, ryb, rzb = rows[q]
                    dot = (rxb * qx + ryb * qy) + rzb * qz
                    d = (rsq + sqc) - 2.0 * dot
                    d = jnp.where((ii >= sr) & (ii < er), d, INF)
                    dbuf[pl.ds(off + q * N, L)] = d
                    msc[q] = jnp.minimum(msc[q], d)
            cpos = s * L + _iota()
            out = []
            for q in range(G):
                cmbuf[pl.ds(pl.multiple_of(s * L, L) + q * NCH, L)] = msc[q]
                upd = msc[q] < mcm[q]
                out.append(jnp.where(upd, msc[q], mcm[q]))
                out.append(jnp.where(upd, cpos, pcm[q]))
            return tuple(out)

        init = (jnp.full((L,), INF, jnp.float32),
                jnp.full((L,), BIGI, jnp.int32)) * G
        st = lax.fori_loop(sc0, sc1, phase_a, init)

        def extract(t, carry):
            out = []
            for q in range(G):
                mcm, pcm, a0, a1 = carry[q * 4:q * 4 + 4]
                g = _vmin_all(mcm)
                spos = _vmin_all(jnp.where(mcm == g, pcm, BIGI))
                lcm = spos & 15
                cidx = (spos >> 4) * 256 + _iota() * L + lcm
                dd = plsc.load_gather(dbuf, [cidx + q * N])
                gp = _vmin_all(jnp.where(dd == g, cidx, BIGI))
                a0 = jnp.where(_iota() == t, gp, a0)
                a1 = jnp.where(_iota() == (t - 16), gp, a1)
                dd = jnp.where(cidx == gp, INF, dd)
                plsc.store_scatter(dbuf, [cidx + q * N], dd)
                nc = _vmin_all(dd)
                plsc.store_scatter(cmbuf, [spos + q * NCH], nc,
                                   mask=_iota() == 0)

                rm = jnp.full((L,), INF, jnp.float32)
                rpos = jnp.full((L,), BIGI, jnp.int32)
                for gi in range(2):
                    scs = sc0 + gi * L + _iota()
                    ok = scs < sc1
                    idx = jnp.where(ok, scs * L + lcm, 0)
                    col = plsc.load_gather(cmbuf, [idx + q * NCH])
                    col = jnp.where(ok, col, INF)
                    upd = col < rm
                    rm = jnp.where(upd, col, rm)
                    rpos = jnp.where(upd, idx, rpos)
                newm = _vmin_all(rm)
                newp = _vmin_all(jnp.where(rm == newm, rpos, BIGI))
                lmask = _iota() == lcm
                out.append(jnp.where(lmask, newm, mcm))
                out.append(jnp.where(lmask, newp, pcm))
                out.append(a0)
                out.append(a1)
            return tuple(out)

        z = jnp.zeros((L,), jnp.int32)
        init_ex = tuple(v for q in range(G)
                        for v in (st[2 * q], st[2 * q + 1], z, z))
        fin = lax.fori_loop(0, K, extract, init_ex)

        for q in range(G):
            roff = pl.multiple_of((i * G + q) * OUTW, OUTW)
            outv[pl.ds(roff, L)] = fin[q * 4 + 2]
            outv[pl.ds(roff + L, L)] = fin[q * 4 + 3]
        return 0

    lax.fori_loop(0, RPW // G, row_body, 0)

    pltpu.sync_copy(outv, outh.at[pl.ds(wid * RPW * OUTW, RPW * OUTW)])


def _sc_call(px, py, pz, b32):
    mesh = plsc.VectorSubcoreMesh(core_axis_name="c", subcore_axis_name="s",
                                  num_cores=2, num_subcores=16)
    fn = pl.kernel(
        _body,
        out_type=jax.ShapeDtypeStruct((N * OUTW,), jnp.int32),
        mesh=mesh,
        compiler_params=pltpu.CompilerParams(needs_layout_passes=False),
        scratch_types=[
            pltpu.VMEM((N + L,), jnp.int32),
            pltpu.VMEM((N + L,), jnp.float32),
            pltpu.VMEM((N + L,), jnp.float32),
            pltpu.VMEM((N + L,), jnp.float32),
            pltpu.VMEM((N + L,), jnp.float32),
            pltpu.VMEM((N + L,), jnp.int32),
            pltpu.VMEM((N + L,), jnp.int32),
            pltpu.VMEM((G * N,), jnp.float32),
            pltpu.VMEM((G * NCH,), jnp.float32),
            pltpu.VMEM((RPW * OUTW,), jnp.int32),
        ],
    )
    return fn(px, py, pz, b32)


def kernel(x, batch):
    xf = x.astype(jnp.float32)
    px = xf[:, 0]
    py = xf[:, 1]
    pz = xf[:, 2]
    b32 = batch.astype(jnp.int32)
    out = _sc_call(px, py, pz, b32)
    return out.reshape(N, OUTW)[:, :K]
